# baseline (speedup 1.0000x reference)
import functools

import jax
import jax.numpy as jnp
from jax import lax
from jax.experimental import pallas as pl
from jax.experimental.pallas import tpu as pltpu
from jax.experimental.pallas import tpu_sc as plsc

_B, _S, _D = 4, 2048, 128
_N = _B * _S
_NC, _NS = 2, 16
_NW = _NC * _NS
_CHUNK = _N // _NW
_WPB = _S // _CHUNK
_GCH = 64
_NG = _CHUNK // _GCH
_PCH = 128
_NP = _CHUNK // _PCH

_mesh = plsc.VectorSubcoreMesh(core_axis_name="c", subcore_axis_name="s")


@functools.partial(
    pl.kernel,
    mesh=_mesh,
    out_type=jax.ShapeDtypeStruct((_B, _S, _D), jnp.float32),
    scratch_types=[
        pltpu.VMEM((_CHUNK,), jnp.int32),
        pltpu.VMEM((_CHUNK, _D), jnp.float32),
        pltpu.SemaphoreType.DMA,
        pltpu.SemaphoreType.DMA,
        pltpu.SemaphoreType.DMA,
        pltpu.SemaphoreType.DMA,
        pltpu.SemaphoreType.DMA,
        pltpu.SemaphoreType.DMA,
        pltpu.SemaphoreType.DMA,
        pltpu.SemaphoreType.DMA,
    ],
)
def _emb(x_hbm, table_hbm, pos_hbm, out_hbm,
         idx_v, rows_v, isem, ps0, ps1, gs0, gs1, gs2, gs3, ws):
    wid = lax.axis_index("s") * _NC + lax.axis_index("c")
    b = wid // _WPB
    s0 = (wid % _WPB) * _CHUNK
    psems = (ps0, ps1)
    gsems = (gs0, gs1, gs2, gs3)

    icp = pltpu.async_copy(x_hbm.at[b, pl.ds(s0, _CHUNK)], idx_v, isem)
    pcps = [
        pltpu.async_copy(
            pos_hbm.at[pl.ds(s0 + p * _PCH, _PCH)],
            rows_v.at[pl.ds(p * _PCH, _PCH)], psems[p])
        for p in range(_NP)
    ]
    icp.wait()

    gcps = []
    for j in range(_NG):
        if j % (_PCH // _GCH) == 0:
            pcps[j // (_PCH // _GCH)].wait()
        gcps.append(pltpu.async_copy(
            table_hbm.at[idx_v.at[pl.ds(j * _GCH, _GCH)]],
            rows_v.at[pl.ds(j * _GCH, _GCH)], gsems[j], add=True))

    wcps = []
    for j in range(_NG):
        gcps[j].wait()
        wcps.append(pltpu.async_copy(
            rows_v.at[pl.ds(j * _GCH, _GCH)],
            out_hbm.at[b, pl.ds(s0 + j * _GCH, _GCH)], ws))
    for cp in wcps:
        cp.wait()


def kernel(x, table, pos_table):
    return _emb(x, table, pos_table)
